# Initial kernel scaffold; baseline (speedup 1.0000x reference)
import jax
import jax.numpy as jnp
from jax.experimental import pallas as pl


def kernel(x, edge_index, W1, b1, W2, b2, W3, b3, Wfc, bfc):
    raise NotImplementedError("write your pallas kernel here")


# speedup vs baseline: 8.6391x; 8.6391x over previous
import functools
import jax
import jax.numpy as jnp
from jax import lax
from jax.experimental import pallas as pl
from jax.experimental.pallas import tpu as pltpu
from jax.experimental.pallas import tpu_sc as plsc

NC = 2
NS = 16
NW = NC * NS
WIN = 128
BLK = 256

@functools.cache
def _mesh():
  return plsc.VectorSubcoreMesh(core_axis_name="c", subcore_axis_name="s",
                                num_cores=NC, num_subcores=NS)


def _sc_degree(n_pad, steps):
  @functools.partial(
      pl.kernel,
      out_type=jax.ShapeDtypeStruct((NW, n_pad), jnp.float32),
      mesh=_mesh(),
      compiler_params=pltpu.CompilerParams(needs_layout_passes=False),
      scratch_types=[
          pltpu.VMEM((n_pad,), jnp.float32),
          pltpu.VMEM((WIN,), jnp.int32),
      ],
  )
  def k(dst_hbm, out_hbm, deg_v, dbuf):
    cid = lax.axis_index("c")
    sid = lax.axis_index("s")
    w = cid * NS + sid
    zeros = jnp.zeros((16,), jnp.float32)
    ones = jnp.ones((16,), jnp.float32)

    @pl.loop(0, n_pad, step=16)
    def _(i):
      deg_v[pl.ds(i, 16)] = zeros

    @pl.loop(0, steps)
    def _(s):
      pltpu.sync_copy(dst_hbm.at[w, s], dbuf)
      for j in range(WIN // 16):
        idx = dbuf[pl.ds(j * 16, 16)]
        plsc.addupdate_scatter(deg_v, [idx], ones)

    pltpu.sync_copy(deg_v, out_hbm.at[w])

  return k


def _sc_scatter_rows(n_pad, steps, h):
  rows_per_sub = n_pad // NS

  @functools.partial(
      pl.kernel,
      out_type=jax.ShapeDtypeStruct((NC, n_pad, h), jnp.float32),
      mesh=_mesh(),
      scratch_types=[
          pltpu.VMEM_SHARED((n_pad, h), jnp.float32),
          pltpu.VMEM((WIN,), jnp.int32),
          pltpu.VMEM((WIN,), jnp.int32),
          pltpu.VMEM((WIN, h), jnp.float32),
          pltpu.VMEM((WIN, h), jnp.float32),
      ],
  )
  def k(g_hbm, src_hbm, dst_hbm, zeros_hbm, out_hbm,
        accum, sbuf, dbuf, rows, zbuf):
    cid = lax.axis_index("c")
    sid = lax.axis_index("s")
    w = cid * NS + sid
    base = sid * rows_per_sub
    pltpu.sync_copy(zeros_hbm, zbuf)
    for kk in range(rows_per_sub // WIN):
      pltpu.sync_copy(zbuf, accum.at[pl.ds(base + kk * WIN, WIN)])
    plsc.subcore_barrier()

    @pl.loop(0, steps)
    def _(s):
      pltpu.sync_copy(src_hbm.at[w, s], sbuf)
      pltpu.sync_copy(dst_hbm.at[w, s], dbuf)
      pltpu.sync_copy(g_hbm.at[sbuf], rows)
      pltpu.sync_copy(rows, accum.at[dbuf], add=True)

    plsc.subcore_barrier()
    for kk in range(rows_per_sub // WIN):
      off = base + kk * WIN
      pltpu.sync_copy(accum.at[pl.ds(off, WIN)], rows)
      pltpu.sync_copy(rows, out_hbm.at[cid, pl.ds(off, WIN)])

  return k


def _elu(x):
  return jnp.where(x > 0, x, jnp.exp(jnp.minimum(x, 0.0)) - 1.0)


def _tc_prep(deg_ref, x_ref, w_ref, g_ref, dinv_ref):
  deg = jnp.sum(deg_ref[...], axis=0)[:, None] + 1.0
  dinv = lax.rsqrt(deg)
  g_ref[...] = jnp.dot(x_ref[...], w_ref[...],
                       preferred_element_type=jnp.float32) * dinv
  dinv_ref[...] = dinv


def _tc_mid(s0_ref, s1_ref, g_ref, dinv_ref, b_ref, w_ref, out_ref):
  dinv = dinv_ref[...]
  a = _elu(dinv * (s0_ref[...] + s1_ref[...] + g_ref[...]) + b_ref[...])
  out_ref[...] = jnp.dot(a, w_ref[...],
                         preferred_element_type=jnp.float32) * dinv


def _tc_final(s0_ref, s1_ref, g_ref, dinv_ref, b_ref, wfc_ref, bfc_ref,
              out_ref):
  dinv = dinv_ref[...]
  a = _elu(dinv * (s0_ref[...] + s1_ref[...] + g_ref[...]) + b_ref[...])
  z = jnp.dot(a, wfc_ref[...], preferred_element_type=jnp.float32) \
      + bfc_ref[...]
  m = jnp.max(z, axis=-1, keepdims=True)
  lse = m + jnp.log(jnp.sum(jnp.exp(z - m), axis=-1, keepdims=True))
  out_ref[...] = z - lse


def _row_spec(h):
  return pl.BlockSpec((BLK, h), lambda i: (i, 0))


def _full_spec(shape):
  return pl.BlockSpec(shape, lambda i: tuple(0 for _ in shape))


@jax.jit
def kernel(x, edge_index, W1, b1, W2, b2, W3, b3, Wfc, bfc):
  n, f_in = x.shape
  h = W1.shape[1]
  c = Wfc.shape[1]
  e = edge_index.shape[1]

  n_pad = ((n + BLK - 1) // BLK) * BLK
  epw = ((e + NW * WIN - 1) // (NW * WIN)) * WIN
  steps = epw // WIN
  e_pad = epw * NW

  x_pad = jnp.zeros((n_pad, f_in), x.dtype).at[:n].set(x)
  pad_idx = jnp.full((e_pad - e,), n, jnp.int32)
  src = jnp.concatenate([edge_index[0], pad_idx]).reshape(NW, steps, WIN)
  dst = jnp.concatenate([edge_index[1], pad_idx]).reshape(NW, steps, WIN)

  zeros_h = jnp.zeros((WIN, h), jnp.float32)

  grid = n_pad // BLK

  deg = _sc_degree(n_pad, steps)(dst)

  g1, dinv = pl.pallas_call(
      _tc_prep,
      grid=(grid,),
      in_specs=[pl.BlockSpec((NW, BLK), lambda i: (0, i)), _row_spec(f_in),
                _full_spec((f_in, h))],
      out_specs=[_row_spec(h), pl.BlockSpec((BLK, 1), lambda i: (i, 0))],
      out_shape=[jax.ShapeDtypeStruct((n_pad, h), jnp.float32),
                 jax.ShapeDtypeStruct((n_pad, 1), jnp.float32)],
  )(deg, x_pad, W1)

  scat = _sc_scatter_rows(n_pad, steps, h)

  def mid(g_prev, b_prev, w_next):
    s = scat(g_prev, src, dst, zeros_h)
    return pl.pallas_call(
        _tc_mid,
        grid=(grid,),
        in_specs=[_row_spec(h), _row_spec(h), _row_spec(h),
                  pl.BlockSpec((BLK, 1), lambda i: (i, 0)),
                  _full_spec((1, h)), _full_spec((h, h))],
        out_specs=_row_spec(h),
        out_shape=jax.ShapeDtypeStruct((n_pad, h), jnp.float32),
    )(s[0], s[1], g_prev, dinv, b_prev.reshape(1, h), w_next)

  g2 = mid(g1, b1, W2)
  g3 = mid(g2, b2, W3)

  s3 = scat(g3, src, dst, zeros_h)
  out = pl.pallas_call(
      _tc_final,
      grid=(grid,),
      in_specs=[_row_spec(h), _row_spec(h), _row_spec(h),
                pl.BlockSpec((BLK, 1), lambda i: (i, 0)),
                _full_spec((1, h)), _full_spec((h, c)), _full_spec((1, c))],
      out_specs=_row_spec(c),
      out_shape=jax.ShapeDtypeStruct((n_pad, c), jnp.float32),
  )(s3[0], s3[1], g3, dinv, b3.reshape(1, h), Wfc, bfc.reshape(1, c))

  return out[:n]
